# initial kernel scaffold (unmeasured)
import jax
import jax.numpy as jnp
from jax import lax
from jax.experimental import pallas as pl
from jax.experimental.pallas import tpu as pltpu


def kernel(
    x,
):
    def body(*refs):
        pass

    out_shape = jax.ShapeDtypeStruct(..., jnp.float32)
    return pl.pallas_call(body, out_shape=out_shape)(...)


# baseline (device time: 35481 ns/iter reference)
import jax
import jax.numpy as jnp
from jax import lax
from jax.experimental import pallas as pl
from jax.experimental.pallas import tpu as pltpu


def kernel(x):
    m, n = x.shape

    def body(x_ref, out_ref, comm_ref, send_sem, recv_sem):
        my_x = lax.axis_index("x")
        my_y = lax.axis_index("y")
        my_z = lax.axis_index("z")
        partner = (1 - my_x, my_y, my_z)

        rdma = pltpu.make_async_remote_copy(
            src_ref=x_ref,
            dst_ref=comm_ref,
            send_sem=send_sem,
            recv_sem=recv_sem,
            device_id=partner,
            device_id_type=pl.DeviceIdType.MESH,
        )
        rdma.start()
        rdma.wait()
        out_ref[...] = x_ref[...] + comm_ref[...]

    return pl.pallas_call(
        body,
        out_shape=jax.ShapeDtypeStruct((m, n), x.dtype),
        in_specs=[pl.BlockSpec(memory_space=pltpu.VMEM)],
        out_specs=pl.BlockSpec(memory_space=pltpu.VMEM),
        scratch_shapes=[
            pltpu.VMEM((m, n), x.dtype),
            pltpu.SemaphoreType.DMA,
            pltpu.SemaphoreType.DMA,
        ],
    )(x)


# device time: 29460 ns/iter; 1.2044x vs baseline; 1.2044x over previous
import jax
import jax.numpy as jnp
from jax import lax
from jax.experimental import pallas as pl
from jax.experimental.pallas import tpu as pltpu

N_CHUNK = 8


def kernel(x):
    m, n = x.shape
    ch = m // N_CHUNK

    def body(
        x_ref,
        out_ref,
        p1_ref,
        p1_send,
        p1_recv,
        fwd_send,
        fwd_recv,
        bwd_send,
        bwd_recv,
    ):
        my_x = lax.axis_index("x")
        my_y = lax.axis_index("y")
        my_z = lax.axis_index("z")
        partner = (1 - my_x, my_y, my_z)

        R = jnp.where(my_y == 0, my_z, 7 - my_z)

        def ring_coords(t):
            t = t % N_CHUNK
            ty = jnp.where(t < 4, 0, 1)
            tz = jnp.where(t < 4, t, 7 - t)
            return (my_x, ty, tz)

        nxt = ring_coords(R + 1)
        prv = ring_coords(R + 7)

        barrier = pltpu.get_barrier_semaphore()
        for nbr in (partner, nxt, prv):
            pl.semaphore_signal(
                barrier, inc=1, device_id=nbr,
                device_id_type=pl.DeviceIdType.MESH,
            )
        pl.semaphore_wait(barrier, 3)

        my_off = R * ch
        p1 = pltpu.make_async_remote_copy(
            src_ref=x_ref.at[pl.ds(my_off, ch)],
            dst_ref=p1_ref,
            send_sem=p1_send,
            recv_sem=p1_recv,
            device_id=partner,
            device_id_type=pl.DeviceIdType.MESH,
        )
        p1.start()
        p1.wait()
        out_ref[pl.ds(my_off, ch), :] = (
            x_ref[pl.ds(my_off, ch), :] + p1_ref[...]
        )

        def send(chunk_idx, dst_dev, send_sem, recv_sem):
            off = (chunk_idx % N_CHUNK) * ch
            r = pltpu.make_async_remote_copy(
                src_ref=out_ref.at[pl.ds(off, ch)],
                dst_ref=out_ref.at[pl.ds(off, ch)],
                send_sem=send_sem,
                recv_sem=recv_sem,
                device_id=dst_dev,
                device_id_type=pl.DeviceIdType.MESH,
            )
            r.start()
            return r

        def wait_recv(chunk_idx, recv_sem):
            off = (chunk_idx % N_CHUNK) * ch
            r = pltpu.make_async_remote_copy(
                src_ref=out_ref.at[pl.ds(off, ch)],
                dst_ref=out_ref.at[pl.ds(off, ch)],
                send_sem=p1_send,
                recv_sem=recv_sem,
                device_id=partner,
                device_id_type=pl.DeviceIdType.MESH,
            )
            r.wait_recv()

        sends = [
            send(R, nxt, fwd_send.at[0], fwd_recv.at[0]),
            send(R, prv, bwd_send.at[0], bwd_recv.at[0]),
        ]
        for h in range(1, 4):
            wait_recv(R - h, fwd_recv.at[h - 1])
            sends.append(send(R - h, nxt, fwd_send.at[h], fwd_recv.at[h]))
            if h <= 2:
                wait_recv(R + h, bwd_recv.at[h - 1])
                sends.append(
                    send(R + h, prv, bwd_send.at[h], bwd_recv.at[h])
                )
        wait_recv(R - 4, fwd_recv.at[3])
        wait_recv(R + 3, bwd_recv.at[2])
        for s in sends:
            s.wait_send()

    return pl.pallas_call(
        body,
        out_shape=jax.ShapeDtypeStruct((m, n), x.dtype),
        in_specs=[pl.BlockSpec(memory_space=pltpu.VMEM)],
        out_specs=pl.BlockSpec(memory_space=pltpu.VMEM),
        scratch_shapes=[
            pltpu.VMEM((ch, n), x.dtype),
            pltpu.SemaphoreType.DMA,
            pltpu.SemaphoreType.DMA,
            pltpu.SemaphoreType.DMA((4,)),
            pltpu.SemaphoreType.DMA((4,)),
            pltpu.SemaphoreType.DMA((3,)),
            pltpu.SemaphoreType.DMA((3,)),
        ],
        compiler_params=pltpu.CompilerParams(collective_id=0),
    )(x)


# device time: 19094 ns/iter; 1.8582x vs baseline; 1.5429x over previous
import jax
import jax.numpy as jnp
from jax import lax
from jax.experimental import pallas as pl
from jax.experimental.pallas import tpu as pltpu

N_CHUNK = 8

X_ORDER = (2, 5, 0, 7)
RING_OFFS = (2, 5)


def kernel(x):
    m, n = x.shape
    ch = m // N_CHUNK

    def body(
        x_ref,
        out_ref,
        p1_ref,
        p1_send,
        p1_recv,
        f_send,
        f_recv,
        b_send,
        b_recv,
    ):
        my_x = lax.axis_index("x")
        my_y = lax.axis_index("y")
        my_z = lax.axis_index("z")
        partner = (1 - my_x, my_y, my_z)

        R = jnp.where(my_y == 0, my_z, 7 - my_z)

        def ring_coords(t):
            t = t % N_CHUNK
            ty = jnp.where(t < 4, 0, 1)
            tz = jnp.where(t < 4, t, 7 - t)
            return (my_x, ty, tz)

        nxt = ring_coords(R + 1)
        prv = ring_coords(R + 7)

        def chunk_off(idx):
            return (idx % N_CHUNK) * ch

        barrier = pltpu.get_barrier_semaphore()
        for nbr in (partner, nxt, prv):
            pl.semaphore_signal(
                barrier, inc=1, device_id=nbr,
                device_id_type=pl.DeviceIdType.MESH,
            )
        pl.semaphore_wait(barrier, 3)

        x_rdmas = []
        for i, s in enumerate(X_ORDER):
            off = chunk_off(R + s)
            r = pltpu.make_async_remote_copy(
                src_ref=x_ref.at[pl.ds(off, ch)],
                dst_ref=p1_ref.at[i],
                send_sem=p1_send.at[i],
                recv_sem=p1_recv.at[i],
                device_id=partner,
                device_id_type=pl.DeviceIdType.MESH,
            )
            r.start()
            x_rdmas.append(r)

        ring_rdmas = []
        for i, s in enumerate(X_ORDER):
            off = chunk_off(R + s)
            x_rdmas[i].wait_recv()
            out_ref[pl.ds(off, ch), :] = (
                x_ref[pl.ds(off, ch), :] + p1_ref[i]
            )
            if s in RING_OFFS:
                j = RING_OFFS.index(s)
                for dst, ssem, rsem in (
                    (nxt, f_send.at[j], f_recv.at[j]),
                    (prv, b_send.at[j], b_recv.at[j]),
                ):
                    r = pltpu.make_async_remote_copy(
                        src_ref=out_ref.at[pl.ds(off, ch)],
                        dst_ref=out_ref.at[pl.ds(off, ch)],
                        send_sem=ssem,
                        recv_sem=rsem,
                        device_id=dst,
                        device_id_type=pl.DeviceIdType.MESH,
                    )
                    r.start()
                    ring_rdmas.append(r)

        def wait_ring_recv(idx, rsem):
            off = chunk_off(idx)
            r = pltpu.make_async_remote_copy(
                src_ref=out_ref.at[pl.ds(off, ch)],
                dst_ref=out_ref.at[pl.ds(off, ch)],
                send_sem=p1_send.at[0],
                recv_sem=rsem,
                device_id=partner,
                device_id_type=pl.DeviceIdType.MESH,
            )
            r.wait_recv()

        wait_ring_recv(R + 1, f_recv.at[0])
        wait_ring_recv(R + 4, f_recv.at[1])
        wait_ring_recv(R + 3, b_recv.at[0])
        wait_ring_recv(R + 6, b_recv.at[1])

        for r in x_rdmas:
            r.wait_send()
        for r in ring_rdmas:
            r.wait_send()

    return pl.pallas_call(
        body,
        out_shape=jax.ShapeDtypeStruct((m, n), x.dtype),
        in_specs=[pl.BlockSpec(memory_space=pltpu.VMEM)],
        out_specs=pl.BlockSpec(memory_space=pltpu.VMEM),
        scratch_shapes=[
            pltpu.VMEM((4, ch, n), x.dtype),
            pltpu.SemaphoreType.DMA((4,)),
            pltpu.SemaphoreType.DMA((4,)),
            pltpu.SemaphoreType.DMA((2,)),
            pltpu.SemaphoreType.DMA((2,)),
            pltpu.SemaphoreType.DMA((2,)),
            pltpu.SemaphoreType.DMA((2,)),
        ],
        compiler_params=pltpu.CompilerParams(collective_id=0),
    )(x)
